# baseline (speedup 1.0000x reference)
import functools

import jax
import jax.numpy as jnp
from jax import lax
from jax.experimental import pallas as pl
from jax.experimental.pallas import tpu as pltpu
from jax.experimental.pallas import tpu_sc as plsc

_EMB = 256
_HID = 256
_L = 512
_B = 4
_T = 3600
_TP = 3648
_PW = 456
_NW = 32
_ZERO_ROW = _B * _L
_CHUNKS = ((0, 120), (120, 120), (240, 120), (360, 96))
_LAST_VALID = 48


def _layer_norm(h, g, b):
    mu = jnp.mean(h, axis=1, keepdims=True)
    d = h - mu
    var = jnp.mean(d * d, axis=1, keepdims=True)
    return d * lax.rsqrt(var + 1e-5) * g + b


def _shift(h, z):
    prev = jnp.concatenate([z, h[:-1, :]], axis=0)
    nxt = jnp.concatenate([h[1:, :], z], axis=0)
    return prev, nxt


def _tc_body(mel_ref, x_ref, tgt_ref, wk1_ref, b1_ref, g1_ref, be1_ref,
             wk2_ref, b2_ref, g2_ref, be2_ref, lw_ref, lb_ref,
             dur_ref, idx_ref):
    b = pl.program_id(0)
    xb = x_ref[0]

    z = jnp.zeros((1, _EMB), jnp.float32)
    xp, xn = _shift(xb, z)
    h = (jnp.dot(xp, wk1_ref[0], preferred_element_type=jnp.float32)
         + jnp.dot(xb, wk1_ref[1], preferred_element_type=jnp.float32)
         + jnp.dot(xn, wk1_ref[2], preferred_element_type=jnp.float32)
         + b1_ref[...])
    h = jax.nn.relu(_layer_norm(h, g1_ref[...], be1_ref[...]))
    hp, hn = _shift(h, z)
    h = (jnp.dot(hp, wk2_ref[0], preferred_element_type=jnp.float32)
         + jnp.dot(h, wk2_ref[1], preferred_element_type=jnp.float32)
         + jnp.dot(hn, wk2_ref[2], preferred_element_type=jnp.float32)
         + b2_ref[...])
    h = jax.nn.relu(_layer_norm(h, g2_ref[...], be2_ref[...]))
    dur = jax.nn.relu(jnp.dot(h, lw_ref[...],
                              preferred_element_type=jnp.float32)
                      + lb_ref[...])
    dur_ref[0] = dur

    rr = lax.broadcasted_iota(jnp.int32, (_L, _L), 0)
    cc = lax.broadcasted_iota(jnp.int32, (_L, _L), 1)
    ut = (rr <= cc).astype(jnp.float32)
    tgt_row = tgt_ref[0].astype(jnp.float32)
    ends_row = jnp.dot(tgt_row, ut,
                       preferred_element_type=jnp.float32)
    ends_i = ends_row.astype(jnp.int32)
    total = ends_i[0, _L - 1]
    limit = jnp.minimum(total, mel_ref[0, 0])
    base = b * _L
    nblk = 4
    blk = _TP // nblk
    for k in range(nblk):
        t_col = lax.broadcasted_iota(jnp.int32, (blk, 1), 0) + k * blk
        cnt = jnp.sum((ends_i <= t_col).astype(jnp.int32), axis=1,
                      keepdims=True)
        absidx = jnp.where(t_col < limit, cnt + base, _ZERO_ROW)
        idx_ref[0, pl.ds(k * blk, blk)] = absidx


def _run_tc(x, tgt, mel, wk1, b1r, g1r, be1r, wk2, b2r, g2r, be2r, lw, lbr):
    full3 = lambda s: pl.BlockSpec(s, lambda b: (0, 0, 0))
    full2 = lambda s: pl.BlockSpec(s, lambda b: (0, 0))
    return pl.pallas_call(
        _tc_body,
        grid=(_B,),
        in_specs=[
            pl.BlockSpec(memory_space=pltpu.SMEM),
            pl.BlockSpec((1, _L, _EMB), lambda b: (b, 0, 0)),
            pl.BlockSpec((1, 1, _L), lambda b: (b, 0, 0)),
            full3((3, _EMB, _HID)),
            full2((1, _HID)), full2((1, _HID)), full2((1, _HID)),
            full3((3, _HID, _HID)),
            full2((1, _HID)), full2((1, _HID)), full2((1, _HID)),
            full2((_HID, 1)),
            full2((1, 1)),
        ],
        out_specs=[
            pl.BlockSpec((1, _L, 1), lambda b: (b, 0, 0)),
            pl.BlockSpec((1, _TP, 1), lambda b: (b, 0, 0)),
        ],
        out_shape=[
            jax.ShapeDtypeStruct((_B, _L, 1), jnp.float32),
            jax.ShapeDtypeStruct((_B, _TP, 1), jnp.int32),
        ],
    )(mel, x, tgt, wk1, b1r, g1r, be1r, wk2, b2r, g2r, be2r, lw, lbr)


def _sc_body(table_hbm, idx_hbm, out_hbm,
             i0, i1, i2, i3, r0, r1, r2, r3, sem):
    cid = lax.axis_index("c")
    sid = lax.axis_index("s")
    wid = sid * 2 + cid
    b = wid // 8
    slot = wid % 8
    pad_base = wid * _PW
    out_base = b * _T + slot * _PW
    idx_bufs = (i0, i1, i2, i3)
    row_bufs = (r0, r1, r2, r3)
    for c, (off, sz) in enumerate(_CHUNKS):
        pltpu.sync_copy(idx_hbm.at[pl.ds(pad_base + off, sz)], idx_bufs[c])
    waits = [pltpu.async_copy(table_hbm.at[idx_bufs[c]], row_bufs[c], sem)
             for c in range(4)]
    for w in waits:
        w.wait()
    for c, (off, sz) in enumerate(_CHUNKS[:3]):
        pltpu.sync_copy(row_bufs[c], out_hbm.at[pl.ds(out_base + off, sz)])
    off3, sz3 = _CHUNKS[3]
    is_last = slot == 7

    @pl.when(is_last)
    def _():
        pltpu.sync_copy(r3.at[pl.ds(0, _LAST_VALID)],
                        out_hbm.at[pl.ds(out_base + off3, _LAST_VALID)])

    @pl.when(jnp.logical_not(is_last))
    def _():
        pltpu.sync_copy(r3, out_hbm.at[pl.ds(out_base + off3, sz3)])


def _run_sc(table, flat_idx):
    mesh = plsc.VectorSubcoreMesh(core_axis_name="c", subcore_axis_name="s")
    f = functools.partial(
        pl.kernel,
        out_type=jax.ShapeDtypeStruct((_B * _T, _EMB), jnp.float32),
        mesh=mesh,
        scratch_types=[
            pltpu.VMEM((120,), jnp.int32),
            pltpu.VMEM((120,), jnp.int32),
            pltpu.VMEM((120,), jnp.int32),
            pltpu.VMEM((96,), jnp.int32),
            pltpu.VMEM((120, _EMB), jnp.float32),
            pltpu.VMEM((120, _EMB), jnp.float32),
            pltpu.VMEM((120, _EMB), jnp.float32),
            pltpu.VMEM((96, _EMB), jnp.float32),
            pltpu.SemaphoreType.DMA,
        ],
    )(_sc_body)
    return f(table, flat_idx)


def kernel(x, target, mel_max_len, conv1_w, conv1_b, ln1_g, ln1_b,
           conv2_w, conv2_b, ln2_g, ln2_b, lin_w, lin_b):
    mel = jnp.asarray(mel_max_len, jnp.int32).reshape(1, 1)
    tgt = target.astype(jnp.int32).reshape(_B, 1, _L)
    wk1 = jnp.transpose(conv1_w, (2, 1, 0))
    wk2 = jnp.transpose(conv2_w, (2, 1, 0))
    dur3, idx3 = _run_tc(
        x, tgt, mel, wk1,
        conv1_b.reshape(1, _HID), ln1_g.reshape(1, _HID), ln1_b.reshape(1, _HID),
        wk2,
        conv2_b.reshape(1, _HID), ln2_g.reshape(1, _HID), ln2_b.reshape(1, _HID),
        lin_w, lin_b.reshape(1, 1))
    duration = dur3[:, :, 0]
    table = jnp.concatenate(
        [x.reshape(_B * _L, _EMB), jnp.zeros((8, _EMB), x.dtype)], axis=0)
    flat_idx = idx3.reshape(_B * _TP)
    out_flat = _run_sc(table, flat_idx)
    output = out_flat.reshape(_B, _T, _EMB)
    return (output, duration)
